# baseline (speedup 1.0000x reference)
import functools

import jax
import jax.numpy as jnp
import numpy as np
from jax import lax
from jax.experimental import pallas as pl
from jax.experimental.pallas import tpu as pltpu
from jax.experimental.pallas import tpu_sc as plsc

N_RAYS = 8192
N_PTS = 524288
GS = 160
K0_DIM = 12
VIEWPE = 4
WIDTH = 128
INTERVAL = 0.5
ALPHA_INIT = 1e-06
ACT_SHIFT = float(np.log(1.0 / (1.0 - ALPHA_INIT) - 1.0))

NW = 32
PTS_PER_W = N_PTS // NW
CB = 128
NCHUNK = PTS_PER_W // CB

_TAPS = [(0, 0, 0), (0, 0, 1), (0, 1, 0), (0, 1, 1),
         (1, 0, 0), (1, 0, 1), (1, 1, 0), (1, 1, 1)]


def _interp_body(xs, ys, zs, rid, table, vdp, out, xs_v, ys_v, zs_v, rid_v,
                 idx_v, rows_v, vdrows_v, out_v, sem):
    c_ax = lax.axis_index("c")
    s_ax = lax.axis_index("s")
    wid = s_ax * 2 + c_ax
    base0 = wid * PTS_PER_W

    def chunk_body(ci, carry):
        base = base0 + ci * CB
        pltpu.sync_copy(xs.at[pl.ds(base, CB)], xs_v)
        pltpu.sync_copy(ys.at[pl.ds(base, CB)], ys_v)
        pltpu.sync_copy(zs.at[pl.ds(base, CB)], zs_v)
        pltpu.sync_copy(rid.at[pl.ds(base, CB)], rid_v)

        fxs, fys, fzs = [], [], []
        for g in range(CB // 16):
            sl = pl.ds(g * 16, 16)
            x = xs_v[sl]
            y = ys_v[sl]
            z = zs_v[sl]
            px = (x + 1.0) * 0.5 * (GS - 1)
            py = (y + 1.0) * 0.5 * (GS - 1)
            pz = (z + 1.0) * 0.5 * (GS - 1)
            x0 = jnp.clip(px.astype(jnp.int32), 0, GS - 2)
            y0 = jnp.clip(py.astype(jnp.int32), 0, GS - 2)
            z0 = jnp.clip(pz.astype(jnp.int32), 0, GS - 2)
            fxs.append(px - x0.astype(jnp.float32))
            fys.append(py - y0.astype(jnp.float32))
            fzs.append(pz - z0.astype(jnp.float32))
            bi = (z0 * GS + y0) * GS + x0
            for t, (dz, dy, dx) in enumerate(_TAPS):
                idx_v[t, sl] = bi + (dz * GS + dy) * GS + dx

        cps = [pltpu.async_copy(table.at[idx_v.at[t]], rows_v.at[t], sem)
               for t in range(8)]
        cps.append(pltpu.async_copy(vdp.at[rid_v], vdrows_v, sem))
        for cp in cps:
            cp.wait()

        for g in range(CB // 16):
            sl = pl.ds(g * 16, 16)
            fx, fy, fz = fxs[g], fys[g], fzs[g]
            ex = 1.0 - fx
            ey = 1.0 - fy
            ez = 1.0 - fz
            wy0 = ey * ez
            wy1 = fy * ez
            wy2 = ey * fz
            wy3 = fy * fz
            w = [ex * wy0, fx * wy0, ex * wy1, fx * wy1,
                 ex * wy2, fx * wy2, ex * wy3, fx * wy3]
            for p in range(16):
                gp = g * 16 + p
                acc = vdrows_v[gp] + w[0][p] * rows_v[0, gp]
                for t in range(1, 8):
                    acc = acc + w[t][p] * rows_v[t, gp]
                out_v[pl.ds(gp * 16, 16)] = acc

        pltpu.sync_copy(out_v, out.at[pl.ds(base * 16, CB * 16)])
        return carry

    lax.fori_loop(0, NCHUNK, chunk_body, 0)


def _interp_call(xs, ys, zs, rid, table, vdp):
    mesh = plsc.VectorSubcoreMesh(core_axis_name="c", subcore_axis_name="s")
    f = functools.partial(
        pl.kernel,
        out_type=jax.ShapeDtypeStruct((N_PTS * 16,), jnp.float32),
        mesh=mesh,
        compiler_params=pltpu.CompilerParams(use_tc_tiling_on_sc=False),
        scratch_types=[
            pltpu.VMEM((CB,), jnp.float32),
            pltpu.VMEM((CB,), jnp.float32),
            pltpu.VMEM((CB,), jnp.float32),
            pltpu.VMEM((CB,), jnp.int32),
            pltpu.VMEM((8, CB), jnp.int32),
            pltpu.VMEM((8, CB, 16), jnp.float32),
            pltpu.VMEM((CB, 16), jnp.float32),
            pltpu.VMEM((CB * 16,), jnp.float32),
            pltpu.SemaphoreType.DMA,
        ],
    )(_interp_body)
    return f(xs, ys, zs, rid, table, vdp)


def kernel(xyz, viewdirs, ray_id, density_grid, k0_grid, w0, b0, w1, b1, w2, b2):
    tbl = jnp.concatenate([density_grid[0], k0_grid[0]], axis=0).reshape(13, -1)
    tbl = jnp.pad(tbl, ((0, 3), (0, 0))).T
    xyzT = xyz.T
    vdp = jnp.pad(viewdirs, ((0, 0), (13, 0)))

    interp = _interp_call(xyzT[0], xyzT[1], xyzT[2], ray_id, tbl,
                          vdp).reshape(N_PTS, 16)

    density = interp[:, 0]
    k0 = interp[:, 1:13]
    vd = interp[:, 13:16]

    alpha = 1.0 - (1.0 + jnp.exp(density + ACT_SHIFT)) ** (-INTERVAL)
    log1m = jnp.log(jnp.clip(1.0 - alpha, 1e-10, 1.0))
    cum = jnp.cumsum(log1m)
    ecs = jnp.concatenate([jnp.zeros((1,), log1m.dtype), cum[:-1]])
    seg_start = jnp.searchsorted(ray_id, jnp.arange(N_RAYS))
    T = jnp.exp(ecs - ecs[seg_start][ray_id])
    weights = alpha * T
    alphainv_last = jnp.exp(jax.ops.segment_sum(log1m, ray_id, num_segments=N_RAYS))
    freqs = (2.0 ** jnp.arange(VIEWPE)).astype(jnp.float32)
    ang = vd[:, :, None] * freqs
    vd_emb = jnp.concatenate([vd, jnp.sin(ang).reshape(vd.shape[0], -1),
                              jnp.cos(ang).reshape(vd.shape[0], -1)], axis=-1)
    feat = jnp.concatenate([k0, vd_emb], axis=-1)
    h = jax.nn.relu(feat @ w0 + b0)
    h = jax.nn.relu(h @ w1 + b1)
    rgb = jax.nn.sigmoid(h @ w2 + b2)
    rgb_marched = jax.ops.segment_sum(weights[:, None] * rgb, ray_id,
                                      num_segments=N_RAYS) + alphainv_last[:, None] * 1.0
    return (rgb_marched, alphainv_last)
